# baseline (speedup 1.0000x reference)
import functools

import jax
import jax.numpy as jnp
from jax import lax
from jax.experimental import pallas as pl
from jax.experimental.pallas import tpu as pltpu
from jax.experimental.pallas import tpu_sc as plsc

N = 10000
E = 320000
IN_DIM = 128
OUT_DIM = 128
DIM = 256
LAYERS = 4

NW = 32
NS = 16
ET = E + N
EC = 10320
ET_PAD = NW * EC
CE = 16512
NCHUNK = ET_PAD // CE
F = 4
NSLICE = DIM // F
NP = 10240
STRIPE = NP // NS
BN = 2048

_mesh = plsc.VectorSubcoreMesh(core_axis_name="c", subcore_axis_name="s")
_sc_params = pltpu.CompilerParams(
    needs_layout_passes=False, use_tc_tiling_on_sc=False)


def _pre_body(x_ref, w1_ref, b1_ref, gw_ref, asw_ref, adw_ref,
              x0_ref, ht_ref, asrc_ref, adst_ref):
    x0 = jnp.dot(x_ref[...], w1_ref[...],
                 preferred_element_type=jnp.float32) + b1_ref[...]
    x0_ref[...] = x0
    bn = x_ref.shape[0]
    for l in range(LAYERS):
        h = jnp.dot(x0, gw_ref[l], preferred_element_type=jnp.float32)
        h16 = jax.lax.bitcast_convert_type(h.astype(jnp.bfloat16),
                                           jnp.uint16)
        lo = h16[:, :DIM // 2].astype(jnp.int32)
        hi = h16[:, DIM // 2:].astype(jnp.int32)
        ht_ref[l] = jnp.transpose(jnp.bitwise_or(lo, lax.shift_left(hi, 16)))
    a_s = jnp.stack([jnp.dot(gw_ref[l], asw_ref[l],
                             preferred_element_type=jnp.float32)
                     for l in range(LAYERS)], axis=1)
    a_d = jnp.stack([jnp.dot(gw_ref[l], adw_ref[l],
                             preferred_element_type=jnp.float32)
                     for l in range(LAYERS)], axis=1)
    asrc_ref[...] = jnp.dot(x0, a_s, preferred_element_type=jnp.float32)
    adst_ref[...] = jnp.dot(x0, a_d, preferred_element_type=jnp.float32)


def _pre(x, lin1_W, lin1_b, gat_W, att_src, att_dst):
    return pl.pallas_call(
        _pre_body,
        grid=(NP // BN,),
        in_specs=[
            pl.BlockSpec((BN, IN_DIM), lambda i: (i, 0)),
            pl.BlockSpec((IN_DIM, DIM), lambda i: (0, 0)),
            pl.BlockSpec((DIM,), lambda i: (0,)),
            pl.BlockSpec((LAYERS, DIM, DIM), lambda i: (0, 0, 0)),
            pl.BlockSpec((LAYERS, DIM), lambda i: (0, 0)),
            pl.BlockSpec((LAYERS, DIM), lambda i: (0, 0)),
        ],
        out_specs=[
            pl.BlockSpec((BN, DIM), lambda i: (i, 0)),
            pl.BlockSpec((LAYERS, DIM // 2, BN), lambda i: (0, 0, i)),
            pl.BlockSpec((BN, LAYERS), lambda i: (i, 0)),
            pl.BlockSpec((BN, LAYERS), lambda i: (i, 0)),
        ],
        out_shape=[
            jax.ShapeDtypeStruct((NP, DIM), jnp.float32),
            jax.ShapeDtypeStruct((LAYERS, DIM // 2, NP), jnp.int32),
            jax.ShapeDtypeStruct((NP, LAYERS), jnp.float32),
            jax.ShapeDtypeStruct((NP, LAYERS), jnp.float32),
        ],
    )(x, lin1_W, lin1_b, gat_W, att_src, att_dst)


def _off8(x):
    return pl.multiple_of(x, 8)


@functools.partial(
    pl.kernel,
    out_type=[
        jax.ShapeDtypeStruct((LAYERS * ET_PAD,), jnp.float32),
        jax.ShapeDtypeStruct((LAYERS * 2 * NP,), jnp.float32),
    ],
    mesh=_mesh,
    scratch_types=[
        pltpu.VMEM((EC,), jnp.int32),
        pltpu.VMEM((EC,), jnp.float32),
        pltpu.VMEM((N,), jnp.float32),
        pltpu.VMEM((N,), jnp.float32),
        pltpu.VMEM((NP,), jnp.float32),
        pltpu.VMEM((NS, STRIPE), jnp.float32),
        pltpu.VMEM((STRIPE,), jnp.float32),
        pltpu.VMEM_SHARED((NS, NP), jnp.float32),
    ],
    compiler_params=_sc_params,
)
def _edge_denom(ep_hbm, asrc_hbm, adst_hbm, e_hbm, den_hbm,
                ep_v, e_v, asrc_v, adst_v, pden_v, stripe_v, sum_v, den_sh):
    cid = lax.axis_index("c")
    sid = lax.axis_index("s")
    wid = sid * 2 + cid
    base = wid * EC
    pltpu.sync_copy(ep_hbm.at[pl.ds(_off8(base), EC)], ep_v)

    def _layer(l, _):
        pltpu.sync_copy(asrc_hbm.at[pl.ds(_off8(l * NP), N)], asrc_v)
        pltpu.sync_copy(adst_hbm.at[pl.ds(_off8(l * NP), N)], adst_v)

        @plsc.parallel_loop(0, NP // 16, unroll=8)
        def _zero(i):
            pden_v[pl.ds(i * 16, 16)] = jnp.zeros((16,), jnp.float32)

        @plsc.parallel_loop(0, EC // 16, unroll=8)
        def _edge(i):
            epv = ep_v[pl.ds(i * 16, 16)]
            srcv = lax.bitwise_and(epv, 0x3FFF)
            dstv = lax.shift_right_logical(epv, 14)
            a = plsc.load_gather(asrc_v, [srcv])
            b = plsc.load_gather(adst_v, [dstv])
            al = a + b
            al = jnp.where(al > 0, al, al * jnp.float32(0.2))
            gid = base + i * 16 + lax.iota(jnp.int32, 16)
            al = jnp.where(gid < ET, al, jnp.float32(-1e30))
            ev = jnp.exp(al)
            e_v[pl.ds(i * 16, 16)] = ev
            plsc.addupdate_scatter(pden_v, [dstv], ev)

        pltpu.sync_copy(e_v, e_hbm.at[pl.ds(_off8(l * ET_PAD + base), EC)])
        pltpu.sync_copy(pden_v, den_sh.at[sid])
        plsc.subcore_barrier()

        for t in range(NS):
            pltpu.sync_copy(den_sh.at[t, pl.ds(sid * STRIPE, STRIPE)],
                            stripe_v.at[t])

        @plsc.parallel_loop(0, STRIPE // 16, unroll=4)
        def _red(i):
            s = stripe_v[0, pl.ds(i * 16, 16)]
            for t in range(1, NS):
                s = s + stripe_v[t, pl.ds(i * 16, 16)]
            sum_v[pl.ds(i * 16, 16)] = s

        pltpu.sync_copy(
            sum_v,
            den_hbm.at[pl.ds(_off8(l * 2 * NP + cid * NP + sid * STRIPE),
                             STRIPE)])
        plsc.subcore_barrier()
        return 0
    lax.fori_loop(0, LAYERS, _layer, 0)


@functools.partial(
    pl.kernel,
    out_type=jax.ShapeDtypeStruct((LAYERS * ET_PAD,), jnp.float32),
    mesh=_mesh,
    scratch_types=[
        pltpu.VMEM((EC,), jnp.int32),
        pltpu.VMEM((EC,), jnp.float32),
        pltpu.VMEM((NP,), jnp.float32),
        pltpu.VMEM((NP,), jnp.float32),
    ],
    compiler_params=_sc_params,
)
def _edge_coef(ep_hbm, e_hbm, den_hbm, coef_hbm, ep_v, e_v, rden_v, dtmp_v):
    cid = lax.axis_index("c")
    sid = lax.axis_index("s")
    wid = sid * 2 + cid
    base = wid * EC
    pltpu.sync_copy(ep_hbm.at[pl.ds(_off8(base), EC)], ep_v)

    def _layer(l, _):
        pltpu.sync_copy(den_hbm.at[pl.ds(_off8(l * 2 * NP), NP)], rden_v)
        pltpu.sync_copy(den_hbm.at[pl.ds(_off8(l * 2 * NP + NP), NP)], dtmp_v)

        @plsc.parallel_loop(0, NP // 16, unroll=4)
        def _rcp(i):
            d = rden_v[pl.ds(i * 16, 16)] + dtmp_v[pl.ds(i * 16, 16)]
            rden_v[pl.ds(i * 16, 16)] = jnp.float32(1.0) / (
                d + jnp.float32(1e-16))

        pltpu.sync_copy(e_hbm.at[pl.ds(_off8(l * ET_PAD + base), EC)], e_v)

        @plsc.parallel_loop(0, EC // 16, unroll=8)
        def _it(i):
            epv = ep_v[pl.ds(i * 16, 16)]
            dstv = lax.shift_right_logical(epv, 14)
            rd = plsc.load_gather(rden_v, [dstv])
            e_v[pl.ds(i * 16, 16)] = e_v[pl.ds(i * 16, 16)] * rd

        pltpu.sync_copy(e_v,
                        coef_hbm.at[pl.ds(_off8(l * ET_PAD + base), EC)])
        return 0
    lax.fori_loop(0, LAYERS, _layer, 0)


@functools.partial(
    pl.kernel,
    out_type=jax.ShapeDtypeStruct((LAYERS * DIM * NP,), jnp.float32),
    mesh=_mesh,
    scratch_types=[
        pltpu.VMEM((2 * NP,), jnp.int32),
        pltpu.VMEM((F * NP,), jnp.float32),
        pltpu.VMEM((CE,), jnp.int32),
        pltpu.VMEM((CE,), jnp.int32),
        pltpu.VMEM((CE,), jnp.float32),
        pltpu.VMEM((CE,), jnp.float32),
        pltpu.SemaphoreType.DMA,
        pltpu.SemaphoreType.DMA,
    ],
    compiler_params=_sc_params,
)
def _edge_aggregate(ep_hbm, coef_hbm, h_hbm, zeros_hbm, g_hbm,
                    hs_v, acc_v, ep_c0, ep_c1, c_c0, c_c1, sem0, sem1):
    cid = lax.axis_index("c")
    sid = lax.axis_index("s")
    wid = sid * 2 + cid
    ep_bufs = (ep_c0, ep_c1)
    c_bufs = (c_c0, c_c1)
    sems = (sem0, sem1)
    for p in range(2):
        sl = wid + NW * p

        def _layer(l, _):
            pltpu.sync_copy(
                h_hbm.at[pl.ds(_off8((l * (DIM // 2) + sl * 2) * NP),
                               2 * NP)], hs_v)
            pltpu.sync_copy(zeros_hbm, acc_v)

            for b in range(2):
                pltpu.async_copy(ep_hbm.at[pl.ds(_off8(b * CE), CE)],
                                 ep_bufs[b], sems[b])
                pltpu.async_copy(
                    coef_hbm.at[pl.ds(_off8(l * ET_PAD + b * CE), CE)],
                    c_bufs[b], sems[b])

            def _chunk2(c2, _):
                for b in range(2):
                    ch = c2 * 2 + b
                    pltpu.make_async_copy(ep_hbm.at[pl.ds(0, CE)],
                                          ep_bufs[b], sems[b]).wait()
                    pltpu.make_async_copy(coef_hbm.at[pl.ds(0, CE)],
                                          c_bufs[b], sems[b]).wait()
                    ep_c = ep_bufs[b]
                    c_c = c_bufs[b]

                    @plsc.parallel_loop(0, CE // 16, unroll=16)
                    def _it(i):
                        epv = ep_c[pl.ds(i * 16, 16)]
                        srcv = lax.bitwise_and(epv, 0x3FFF)
                        dstv = lax.shift_right_logical(epv, 14)
                        cf = c_c[pl.ds(i * 16, 16)]
                        for p2 in range(2):
                            g = plsc.load_gather(hs_v, [srcv + p2 * NP])
                            h_lo = plsc.bitcast(
                                lax.shift_left(g, 16), jnp.float32)
                            h_hi = plsc.bitcast(
                                lax.bitwise_and(g, jnp.int32(-65536)),
                                jnp.float32)
                            plsc.addupdate_scatter(
                                acc_v, [dstv + p2 * NP], h_lo * cf)
                            plsc.addupdate_scatter(
                                acc_v, [dstv + (2 + p2) * NP], h_hi * cf)

                    nxt = ch + 2
                    @pl.when(nxt < NCHUNK)
                    def _():
                        pltpu.async_copy(
                            ep_hbm.at[pl.ds(_off8(nxt * CE), CE)],
                            ep_bufs[b], sems[b])
                        pltpu.async_copy(
                            coef_hbm.at[
                                pl.ds(_off8(l * ET_PAD + nxt * CE), CE)],
                            c_bufs[b], sems[b])
                return 0
            lax.fori_loop(0, NCHUNK // 2, _chunk2, 0)

            pltpu.sync_copy(
                acc_v.at[pl.ds(0, 2 * NP)],
                g_hbm.at[pl.ds(_off8((l * DIM + sl * 2) * NP), 2 * NP)])
            pltpu.sync_copy(
                acc_v.at[pl.ds(2 * NP, 2 * NP)],
                g_hbm.at[pl.ds(_off8((l * DIM + DIM // 2 + sl * 2) * NP),
                               2 * NP)])
            return 0
        lax.fori_loop(0, LAYERS, _layer, 0)


def _post_body(g_ref, gb_ref, x0_ref, wih_ref, whh_ref, w2_ref, b2_ref,
               out_ref):
    xc = x0_ref[...]
    h = jnp.zeros((BN, DIM), jnp.float32)
    c = jnp.zeros((BN, DIM), jnp.float32)
    cdims = (((1,), (1,)), ((), ()))
    for l in range(LAYERS):
        ht = jnp.tanh(jnp.transpose(g_ref[l]) + gb_ref[l])
        gates = (lax.dot_general(ht, wih_ref[l, :, :DIM], cdims,
                                 preferred_element_type=jnp.float32)
                 + lax.dot_general(xc, wih_ref[l, :, DIM:], cdims,
                                   preferred_element_type=jnp.float32)
                 + lax.dot_general(h, whh_ref[l], cdims,
                                   preferred_element_type=jnp.float32))
        ig = jax.nn.sigmoid(gates[:, :DIM])
        fg = jax.nn.sigmoid(gates[:, DIM:2 * DIM])
        gg = jnp.tanh(gates[:, 2 * DIM:3 * DIM])
        og = jax.nn.sigmoid(gates[:, 3 * DIM:])
        c = fg * c + ig * gg
        h = og * jnp.tanh(c)
        xc = h
    out_ref[...] = jnp.dot(xc, w2_ref[...],
                           preferred_element_type=jnp.float32) + b2_ref[...]


def _post(G, gat_b, x0, Wih, Whh, lin2_W, lin2_b):
    return pl.pallas_call(
        _post_body,
        grid=(NP // BN,),
        in_specs=[
            pl.BlockSpec((LAYERS, DIM, BN), lambda i: (0, 0, i)),
            pl.BlockSpec((LAYERS, DIM), lambda i: (0, 0)),
            pl.BlockSpec((BN, DIM), lambda i: (i, 0)),
            pl.BlockSpec((LAYERS, 4 * DIM, 2 * DIM), lambda i: (0, 0, 0)),
            pl.BlockSpec((LAYERS, 4 * DIM, DIM), lambda i: (0, 0, 0)),
            pl.BlockSpec((DIM, OUT_DIM), lambda i: (0, 0)),
            pl.BlockSpec((OUT_DIM,), lambda i: (0,)),
        ],
        out_specs=pl.BlockSpec((BN, OUT_DIM), lambda i: (i, 0)),
        out_shape=jax.ShapeDtypeStruct((NP, OUT_DIM), jnp.float32),
    )(G, gat_b, x0, Wih, Whh, lin2_W, lin2_b)


def kernel(x, edge_index, lin1_W, lin1_b, gat_W, att_src, att_dst, gat_b,
           lstm_Wih, lstm_Whh, lin2_W, lin2_b):
    ei = edge_index.astype(jnp.int32)
    loop = jnp.arange(N, dtype=jnp.int32)
    pad = jnp.zeros((ET_PAD - ET,), jnp.int32)
    src = jnp.concatenate([ei[0], loop, pad])
    dst = jnp.concatenate([ei[1], loop, pad])
    ep = jnp.bitwise_or(src, jnp.left_shift(dst, 14))

    x0, HT, asrc, adst = _pre(x, lin1_W, lin1_b, gat_W, att_src, att_dst)

    e, den = _edge_denom(ep, asrc.T.reshape(-1), adst.T.reshape(-1))
    coef = _edge_coef(ep, e, den)
    zeros = jnp.zeros((F * NP,), jnp.float32)
    G = _edge_aggregate(ep, coef, HT.reshape(-1), zeros)

    out = _post(G.reshape(LAYERS, DIM, NP), gat_b, x0,
                lstm_Wih, lstm_Whh, lin2_W, lin2_b)
    return out[:N]


# speedup vs baseline: 1.0067x; 1.0067x over previous
import functools

import jax
import jax.numpy as jnp
from jax import lax
from jax.experimental import pallas as pl
from jax.experimental.pallas import tpu as pltpu
from jax.experimental.pallas import tpu_sc as plsc

N = 10000
E = 320000
IN_DIM = 128
OUT_DIM = 128
DIM = 256
LAYERS = 4

NW = 32
NS = 16
ET = E + N
EC = 10320
ET_PAD = NW * EC
CE = 10320
NCHUNK = ET_PAD // CE
F = 4
NSLICE = DIM // F
NP = 10240
STRIPE = NP // NS
BN = 2048

_mesh = plsc.VectorSubcoreMesh(core_axis_name="c", subcore_axis_name="s")
_sc_params = pltpu.CompilerParams(
    needs_layout_passes=False, use_tc_tiling_on_sc=False)


def _pre_a_body(x_ref, w1_ref, b1_ref, gw_ref, asw_ref, adw_ref,
                x0_ref, asrc_ref, adst_ref):
    x0 = jnp.dot(x_ref[...], w1_ref[...],
                 preferred_element_type=jnp.float32) + b1_ref[...]
    x0_ref[...] = x0
    a_s = jnp.stack([jnp.dot(gw_ref[l], asw_ref[l],
                             preferred_element_type=jnp.float32)
                     for l in range(LAYERS)], axis=1)
    a_d = jnp.stack([jnp.dot(gw_ref[l], adw_ref[l],
                             preferred_element_type=jnp.float32)
                     for l in range(LAYERS)], axis=1)
    asrc_ref[...] = jnp.dot(x0, a_s, preferred_element_type=jnp.float32)
    adst_ref[...] = jnp.dot(x0, a_d, preferred_element_type=jnp.float32)


def _pre_a(x, lin1_W, lin1_b, gat_W, att_src, att_dst):
    return pl.pallas_call(
        _pre_a_body,
        grid=(NP // BN,),
        in_specs=[
            pl.BlockSpec((BN, IN_DIM), lambda i: (i, 0)),
            pl.BlockSpec((IN_DIM, DIM), lambda i: (0, 0)),
            pl.BlockSpec((DIM,), lambda i: (0,)),
            pl.BlockSpec((LAYERS, DIM, DIM), lambda i: (0, 0, 0)),
            pl.BlockSpec((LAYERS, DIM), lambda i: (0, 0)),
            pl.BlockSpec((LAYERS, DIM), lambda i: (0, 0)),
        ],
        out_specs=[
            pl.BlockSpec((BN, DIM), lambda i: (i, 0)),
            pl.BlockSpec((BN, LAYERS), lambda i: (i, 0)),
            pl.BlockSpec((BN, LAYERS), lambda i: (i, 0)),
        ],
        out_shape=[
            jax.ShapeDtypeStruct((NP, DIM), jnp.float32),
            jax.ShapeDtypeStruct((NP, LAYERS), jnp.float32),
            jax.ShapeDtypeStruct((NP, LAYERS), jnp.float32),
        ],
    )(x, lin1_W, lin1_b, gat_W, att_src, att_dst)


def _pre_b_body(x0_ref, gw_ref, ht_ref):
    x0 = x0_ref[...]
    bn = x0.shape[0]
    for l in range(LAYERS):
        h = jnp.dot(x0, gw_ref[l], preferred_element_type=jnp.float32)
        h16 = jax.lax.bitcast_convert_type(h.astype(jnp.bfloat16),
                                           jnp.uint16)
        lo = h16[:, :DIM // 2].astype(jnp.int32)
        hi = h16[:, DIM // 2:].astype(jnp.int32)
        ht_ref[l] = jnp.transpose(jnp.bitwise_or(lo, lax.shift_left(hi, 16)))


def _pre_b(x0, gat_W):
    return pl.pallas_call(
        _pre_b_body,
        grid=(NP // BN,),
        in_specs=[
            pl.BlockSpec((BN, DIM), lambda i: (i, 0)),
            pl.BlockSpec((LAYERS, DIM, DIM), lambda i: (0, 0, 0)),
        ],
        out_specs=pl.BlockSpec((LAYERS, DIM // 2, BN), lambda i: (0, 0, i)),
        out_shape=jax.ShapeDtypeStruct((LAYERS, DIM // 2, NP), jnp.int32),
    )(x0, gat_W)


def _off8(x):
    return pl.multiple_of(x, 8)


@functools.partial(
    pl.kernel,
    out_type=[
        jax.ShapeDtypeStruct((LAYERS * ET_PAD,), jnp.float32),
        jax.ShapeDtypeStruct((LAYERS * 2 * NP,), jnp.float32),
    ],
    mesh=_mesh,
    scratch_types=[
        pltpu.VMEM((EC,), jnp.int32),
        pltpu.VMEM((EC,), jnp.float32),
        pltpu.VMEM((N,), jnp.float32),
        pltpu.VMEM((N,), jnp.float32),
        pltpu.VMEM((NP,), jnp.float32),
        pltpu.VMEM((NS, STRIPE), jnp.float32),
        pltpu.VMEM((STRIPE,), jnp.float32),
        pltpu.VMEM_SHARED((NS, NP), jnp.float32),
    ],
    compiler_params=_sc_params,
)
def _edge_denom(ep_hbm, asrc_hbm, adst_hbm, e_hbm, den_hbm,
                ep_v, e_v, asrc_v, adst_v, pden_v, stripe_v, sum_v, den_sh):
    cid = lax.axis_index("c")
    sid = lax.axis_index("s")
    wid = sid * 2 + cid
    base = wid * EC
    pltpu.sync_copy(ep_hbm.at[pl.ds(_off8(base), EC)], ep_v)

    def _layer(l, _):
        pltpu.sync_copy(asrc_hbm.at[pl.ds(_off8(l * NP), N)], asrc_v)
        pltpu.sync_copy(adst_hbm.at[pl.ds(_off8(l * NP), N)], adst_v)

        @plsc.parallel_loop(0, NP // 16, unroll=8)
        def _zero(i):
            pden_v[pl.ds(i * 16, 16)] = jnp.zeros((16,), jnp.float32)

        @plsc.parallel_loop(0, EC // 16, unroll=8)
        def _edge(i):
            epv = ep_v[pl.ds(i * 16, 16)]
            srcv = lax.bitwise_and(epv, 0x3FFF)
            dstv = lax.shift_right_logical(epv, 14)
            a = plsc.load_gather(asrc_v, [srcv])
            b = plsc.load_gather(adst_v, [dstv])
            al = a + b
            al = jnp.where(al > 0, al, al * jnp.float32(0.2))
            gid = base + i * 16 + lax.iota(jnp.int32, 16)
            al = jnp.where(gid < ET, al, jnp.float32(-1e30))
            ev = jnp.exp(al)
            e_v[pl.ds(i * 16, 16)] = ev
            plsc.addupdate_scatter(pden_v, [dstv], ev)

        pltpu.sync_copy(e_v, e_hbm.at[pl.ds(_off8(l * ET_PAD + base), EC)])
        pltpu.sync_copy(pden_v, den_sh.at[sid])
        plsc.subcore_barrier()

        for t in range(NS):
            pltpu.sync_copy(den_sh.at[t, pl.ds(sid * STRIPE, STRIPE)],
                            stripe_v.at[t])

        @plsc.parallel_loop(0, STRIPE // 16, unroll=4)
        def _red(i):
            s = stripe_v[0, pl.ds(i * 16, 16)]
            for t in range(1, NS):
                s = s + stripe_v[t, pl.ds(i * 16, 16)]
            sum_v[pl.ds(i * 16, 16)] = s

        pltpu.sync_copy(
            sum_v,
            den_hbm.at[pl.ds(_off8(l * 2 * NP + cid * NP + sid * STRIPE),
                             STRIPE)])
        plsc.subcore_barrier()
        return 0
    lax.fori_loop(0, LAYERS, _layer, 0)


@functools.partial(
    pl.kernel,
    out_type=jax.ShapeDtypeStruct((LAYERS * ET_PAD,), jnp.float32),
    mesh=_mesh,
    scratch_types=[
        pltpu.VMEM((EC,), jnp.int32),
        pltpu.VMEM((EC,), jnp.float32),
        pltpu.VMEM((NP,), jnp.float32),
        pltpu.VMEM((NP,), jnp.float32),
    ],
    compiler_params=_sc_params,
)
def _edge_coef(ep_hbm, e_hbm, den_hbm, coef_hbm, ep_v, e_v, rden_v, dtmp_v):
    cid = lax.axis_index("c")
    sid = lax.axis_index("s")
    wid = sid * 2 + cid
    base = wid * EC
    pltpu.sync_copy(ep_hbm.at[pl.ds(_off8(base), EC)], ep_v)

    def _layer(l, _):
        pltpu.sync_copy(den_hbm.at[pl.ds(_off8(l * 2 * NP), NP)], rden_v)
        pltpu.sync_copy(den_hbm.at[pl.ds(_off8(l * 2 * NP + NP), NP)], dtmp_v)

        @plsc.parallel_loop(0, NP // 16, unroll=4)
        def _rcp(i):
            d = rden_v[pl.ds(i * 16, 16)] + dtmp_v[pl.ds(i * 16, 16)]
            rden_v[pl.ds(i * 16, 16)] = jnp.float32(1.0) / (
                d + jnp.float32(1e-16))

        pltpu.sync_copy(e_hbm.at[pl.ds(_off8(l * ET_PAD + base), EC)], e_v)

        @plsc.parallel_loop(0, EC // 16, unroll=8)
        def _it(i):
            epv = ep_v[pl.ds(i * 16, 16)]
            dstv = lax.shift_right_logical(epv, 14)
            rd = plsc.load_gather(rden_v, [dstv])
            e_v[pl.ds(i * 16, 16)] = e_v[pl.ds(i * 16, 16)] * rd

        pltpu.sync_copy(e_v,
                        coef_hbm.at[pl.ds(_off8(l * ET_PAD + base), EC)])
        return 0
    lax.fori_loop(0, LAYERS, _layer, 0)


@functools.partial(
    pl.kernel,
    out_type=jax.ShapeDtypeStruct((LAYERS * DIM * NP,), jnp.float32),
    mesh=_mesh,
    scratch_types=[
        pltpu.VMEM((2 * NP,), jnp.int32),
        pltpu.VMEM((F * NP,), jnp.float32),
        pltpu.VMEM((CE,), jnp.int32),
        pltpu.VMEM((CE,), jnp.int32),
        pltpu.VMEM((CE,), jnp.float32),
        pltpu.VMEM((CE,), jnp.float32),
        pltpu.SemaphoreType.DMA,
        pltpu.SemaphoreType.DMA,
    ],
    compiler_params=_sc_params,
)
def _edge_aggregate(ep_hbm, coef_hbm, h_hbm, zeros_hbm, g_hbm,
                    hs_v, acc_v, ep_c0, ep_c1, c_c0, c_c1, sem0, sem1):
    cid = lax.axis_index("c")
    sid = lax.axis_index("s")
    wid = sid * 2 + cid
    ep_bufs = (ep_c0, ep_c1)
    c_bufs = (c_c0, c_c1)
    sems = (sem0, sem1)
    for p in range(2):
        sl = wid + NW * p

        def _layer(l, _):
            pltpu.sync_copy(
                h_hbm.at[pl.ds(_off8((l * (DIM // 2) + sl * 2) * NP),
                               2 * NP)], hs_v)
            pltpu.sync_copy(zeros_hbm, acc_v)

            for b in range(2):
                pltpu.async_copy(ep_hbm.at[pl.ds(_off8(b * CE), CE)],
                                 ep_bufs[b], sems[b])
                pltpu.async_copy(
                    coef_hbm.at[pl.ds(_off8(l * ET_PAD + b * CE), CE)],
                    c_bufs[b], sems[b])

            def _chunk2(c2, _):
                for b in range(2):
                    ch = c2 * 2 + b
                    pltpu.make_async_copy(ep_hbm.at[pl.ds(0, CE)],
                                          ep_bufs[b], sems[b]).wait()
                    pltpu.make_async_copy(coef_hbm.at[pl.ds(0, CE)],
                                          c_bufs[b], sems[b]).wait()
                    ep_c = ep_bufs[b]
                    c_c = c_bufs[b]

                    @plsc.parallel_loop(0, CE // 16, unroll=16)
                    def _it(i):
                        epv = ep_c[pl.ds(i * 16, 16)]
                        srcv = lax.bitwise_and(epv, 0x3FFF)
                        dstv = lax.shift_right_logical(epv, 14)
                        cf = c_c[pl.ds(i * 16, 16)]
                        for p2 in range(2):
                            g = plsc.load_gather(hs_v, [srcv + p2 * NP])
                            h_lo = plsc.bitcast(
                                lax.shift_left(g, 16), jnp.float32)
                            h_hi = plsc.bitcast(
                                lax.bitwise_and(g, jnp.int32(-65536)),
                                jnp.float32)
                            plsc.addupdate_scatter(
                                acc_v, [dstv + p2 * NP], h_lo * cf)
                            plsc.addupdate_scatter(
                                acc_v, [dstv + (2 + p2) * NP], h_hi * cf)

                    nxt = ch + 2
                    @pl.when(nxt < NCHUNK)
                    def _():
                        pltpu.async_copy(
                            ep_hbm.at[pl.ds(_off8(nxt * CE), CE)],
                            ep_bufs[b], sems[b])
                        pltpu.async_copy(
                            coef_hbm.at[
                                pl.ds(_off8(l * ET_PAD + nxt * CE), CE)],
                            c_bufs[b], sems[b])
                return 0
            lax.fori_loop(0, NCHUNK // 2, _chunk2, 0)

            pltpu.sync_copy(
                acc_v.at[pl.ds(0, 2 * NP)],
                g_hbm.at[pl.ds(_off8((l * DIM + sl * 2) * NP), 2 * NP)])
            pltpu.sync_copy(
                acc_v.at[pl.ds(2 * NP, 2 * NP)],
                g_hbm.at[pl.ds(_off8((l * DIM + DIM // 2 + sl * 2) * NP),
                               2 * NP)])
            return 0
        lax.fori_loop(0, LAYERS, _layer, 0)


def _post_body(g_ref, gb_ref, x0_ref, wih_ref, whh_ref, w2_ref, b2_ref,
               out_ref):
    xc = x0_ref[...]
    h = jnp.zeros((BN, DIM), jnp.float32)
    c = jnp.zeros((BN, DIM), jnp.float32)
    cdims = (((1,), (1,)), ((), ()))
    for l in range(LAYERS):
        ht = jnp.tanh(jnp.transpose(g_ref[l]) + gb_ref[l])
        gates = (lax.dot_general(ht, wih_ref[l, :, :DIM], cdims,
                                 preferred_element_type=jnp.float32)
                 + lax.dot_general(xc, wih_ref[l, :, DIM:], cdims,
                                   preferred_element_type=jnp.float32)
                 + lax.dot_general(h, whh_ref[l], cdims,
                                   preferred_element_type=jnp.float32))
        ig = jax.nn.sigmoid(gates[:, :DIM])
        fg = jax.nn.sigmoid(gates[:, DIM:2 * DIM])
        gg = jnp.tanh(gates[:, 2 * DIM:3 * DIM])
        og = jax.nn.sigmoid(gates[:, 3 * DIM:])
        c = fg * c + ig * gg
        h = og * jnp.tanh(c)
        xc = h
    out_ref[...] = jnp.dot(xc, w2_ref[...],
                           preferred_element_type=jnp.float32) + b2_ref[...]


def _post(G, gat_b, x0, Wih, Whh, lin2_W, lin2_b):
    return pl.pallas_call(
        _post_body,
        grid=(NP // BN,),
        in_specs=[
            pl.BlockSpec((LAYERS, DIM, BN), lambda i: (0, 0, i)),
            pl.BlockSpec((LAYERS, DIM), lambda i: (0, 0)),
            pl.BlockSpec((BN, DIM), lambda i: (i, 0)),
            pl.BlockSpec((LAYERS, 4 * DIM, 2 * DIM), lambda i: (0, 0, 0)),
            pl.BlockSpec((LAYERS, 4 * DIM, DIM), lambda i: (0, 0, 0)),
            pl.BlockSpec((DIM, OUT_DIM), lambda i: (0, 0)),
            pl.BlockSpec((OUT_DIM,), lambda i: (0,)),
        ],
        out_specs=pl.BlockSpec((BN, OUT_DIM), lambda i: (i, 0)),
        out_shape=jax.ShapeDtypeStruct((NP, OUT_DIM), jnp.float32),
    )(G, gat_b, x0, Wih, Whh, lin2_W, lin2_b)


def kernel(x, edge_index, lin1_W, lin1_b, gat_W, att_src, att_dst, gat_b,
           lstm_Wih, lstm_Whh, lin2_W, lin2_b):
    ei = edge_index.astype(jnp.int32)
    loop = jnp.arange(N, dtype=jnp.int32)
    pad = jnp.zeros((ET_PAD - ET,), jnp.int32)
    src = jnp.concatenate([ei[0], loop, pad])
    dst = jnp.concatenate([ei[1], loop, pad])
    ep = jnp.bitwise_or(src, jnp.left_shift(dst, 14))

    x0, asrc, adst = _pre_a(x, lin1_W, lin1_b, gat_W, att_src, att_dst)

    e, den = _edge_denom(ep, asrc.T.reshape(-1), adst.T.reshape(-1))
    HT = _pre_b(x0, gat_W)
    coef = _edge_coef(ep, e, den)
    zeros = jnp.zeros((F * NP,), jnp.float32)
    G = _edge_aggregate(ep, coef, HT.reshape(-1), zeros)

    out = _post(G.reshape(LAYERS, DIM, NP), gat_b, x0,
                lstm_Wih, lstm_Whh, lin2_W, lin2_b)
    return out[:N]
